# Initial kernel scaffold; baseline (speedup 1.0000x reference)
import jax
import jax.numpy as jnp
from jax.experimental import pallas as pl


def kernel(x, edge_index, edge_label_index, pair_feats, Wl1, Wr1, b1, Wl2, Wr2, b2, Wm1, bm1, Wm2, bm2, Wm3, bm3):
    raise NotImplementedError("write your pallas kernel here")


# speedup vs baseline: 4.6975x; 4.6975x over previous
import functools

import jax
import jax.numpy as jnp
from jax import lax
from jax.experimental import pallas as pl
from jax.experimental.pallas import tpu as pltpu
from jax.experimental.pallas import tpu_sc as plsc

_NC = 2
_NS = 16
_NW = _NC * _NS

_SC_PARAMS = pltpu.CompilerParams(use_tc_tiling_on_sc=False)


def _pick_chunk(per_worker: int, cap: int = 128) -> int:
    best = 0
    for b in range(8, cap + 1, 8):
        if per_worker % b == 0:
            best = b
    return best


def _seg_sum_sc(table, src, dst, zeros_d, zeros_1, ones_b, with_deg):
    n, d = table.shape
    e = src.shape[0]
    epw = e // _NW
    b = _pick_chunk(epw)
    steps = epw // b
    npt = n // _NS
    mesh = plsc.VectorSubcoreMesh(core_axis_name="c", subcore_axis_name="s")

    out_type = [jax.ShapeDtypeStruct((_NC, n, d), jnp.float32)]
    scratch = [
        pltpu.VMEM((b,), jnp.int32),
        pltpu.VMEM((b,), jnp.int32),
        pltpu.VMEM((b, d), jnp.float32),
        pltpu.VMEM_SHARED((n, d), jnp.float32),
        pltpu.SemaphoreType.DMA,
    ]
    if with_deg:
        out_type.append(jax.ShapeDtypeStruct((_NC, n, 16), jnp.float32))
        scratch += [
            pltpu.VMEM((b, 16), jnp.float32),
            pltpu.VMEM_SHARED((n, 16), jnp.float32),
        ]

    if with_deg:
        @functools.partial(pl.kernel, out_type=out_type, mesh=mesh,
                           scratch_types=scratch, compiler_params=_SC_PARAMS)
        def k(table_hbm, src_hbm, dst_hbm, zd_hbm, z1_hbm, ones_hbm,
              seg_out, deg_out, idx_s, idx_d, rows, tab_sh, sem,
              ones_v, deg_sh):
            cid = lax.axis_index("c")
            sid = lax.axis_index("s")
            wid = cid * _NS + sid
            r0 = sid * npt
            pltpu.sync_copy(zd_hbm.at[pl.ds(r0, npt)], tab_sh.at[pl.ds(r0, npt)])
            pltpu.sync_copy(z1_hbm.at[pl.ds(r0, npt)], deg_sh.at[pl.ds(r0, npt)])
            pltpu.sync_copy(ones_hbm, ones_v)
            plsc.subcore_barrier()

            def step(j, carry):
                base = wid * epw + j * b
                pltpu.sync_copy(src_hbm.at[pl.ds(base, b)], idx_s)
                pltpu.sync_copy(dst_hbm.at[pl.ds(base, b)], idx_d)
                pltpu.async_copy(table_hbm.at[idx_s], rows, sem).wait()
                pltpu.sync_copy(rows, tab_sh.at[idx_d], add=True)
                pltpu.sync_copy(ones_v, deg_sh.at[idx_d], add=True)
                return carry

            lax.fori_loop(0, steps, step, 0)
            plsc.subcore_barrier()
            pltpu.sync_copy(tab_sh.at[pl.ds(r0, npt)],
                            seg_out.at[cid, pl.ds(r0, npt)])
            pltpu.sync_copy(deg_sh.at[pl.ds(r0, npt)],
                            deg_out.at[cid, pl.ds(r0, npt)])

        return k(table, src, dst, zeros_d, zeros_1, ones_b)

    @functools.partial(pl.kernel, out_type=out_type, mesh=mesh,
                       scratch_types=scratch, compiler_params=_SC_PARAMS)
    def k2(table_hbm, src_hbm, dst_hbm, zd_hbm,
           seg_out, idx_s, idx_d, rows, tab_sh, sem):
        cid = lax.axis_index("c")
        sid = lax.axis_index("s")
        wid = cid * _NS + sid
        r0 = sid * npt
        pltpu.sync_copy(zd_hbm.at[pl.ds(r0, npt)], tab_sh.at[pl.ds(r0, npt)])
        plsc.subcore_barrier()

        def step(j, carry):
            base = wid * epw + j * b
            pltpu.sync_copy(src_hbm.at[pl.ds(base, b)], idx_s)
            pltpu.sync_copy(dst_hbm.at[pl.ds(base, b)], idx_d)
            pltpu.async_copy(table_hbm.at[idx_s], rows, sem).wait()
            pltpu.sync_copy(rows, tab_sh.at[idx_d], add=True)
            return carry

        lax.fori_loop(0, steps, step, 0)
        plsc.subcore_barrier()
        pltpu.sync_copy(tab_sh.at[pl.ds(r0, npt)],
                        seg_out.at[cid, pl.ds(r0, npt)])

    return k2(table, src, dst, zeros_d)[0]


def _pair_gather_sc(z, u_idx, v_idx):
    n, d = z.shape
    pp = u_idx.shape[0]
    ppw = pp // _NW
    b = _pick_chunk(ppw)
    steps = ppw // b
    mesh = plsc.VectorSubcoreMesh(core_axis_name="c", subcore_axis_name="s")

    @functools.partial(
        pl.kernel,
        out_type=[jax.ShapeDtypeStruct((pp, d), jnp.float32),
                  jax.ShapeDtypeStruct((pp, d), jnp.float32)],
        mesh=mesh,
        scratch_types=[
            pltpu.VMEM((b,), jnp.int32),
            pltpu.VMEM((b,), jnp.int32),
            pltpu.VMEM((b, d), jnp.float32),
            pltpu.VMEM((b, d), jnp.float32),
            pltpu.SemaphoreType.DMA,
            pltpu.SemaphoreType.DMA,
        ],
        compiler_params=_SC_PARAMS,
    )
    def k(z_hbm, u_hbm, v_hbm, zu_out, zv_out, iu, iv, ru, rv, s1, s2):
        cid = lax.axis_index("c")
        sid = lax.axis_index("s")
        wid = cid * _NS + sid

        def step(j, carry):
            base = wid * ppw + j * b
            pltpu.sync_copy(u_hbm.at[pl.ds(base, b)], iu)
            pltpu.sync_copy(v_hbm.at[pl.ds(base, b)], iv)
            cu = pltpu.async_copy(z_hbm.at[iu], ru, s1)
            cv = pltpu.async_copy(z_hbm.at[iv], rv, s2)
            cu.wait()
            cv.wait()
            pltpu.sync_copy(ru, zu_out.at[pl.ds(base, b)])
            pltpu.sync_copy(rv, zv_out.at[pl.ds(base, b)])
            return carry

        lax.fori_loop(0, steps, step, 0)

    return k(z, u_idx, v_idx)


def _encode_in(x, wcat):
    n = x.shape[0]
    h2 = wcat.shape[1]
    h = h2 // 2

    def body(x_ref, w_ref, xl_ref, xr_ref):
        xw = jnp.dot(x_ref[...], w_ref[...], preferred_element_type=jnp.float32)
        xl_ref[...] = xw[:, :h]
        xr_ref[...] = xw[:, h:]

    return pl.pallas_call(
        body,
        out_shape=[jax.ShapeDtypeStruct((n, h), jnp.float32),
                   jax.ShapeDtypeStruct((n, h), jnp.float32)],
    )(x, wcat)


def _mid(seg1p, degp, xr, b1r, wcat2):
    n, h = xr.shape
    o2 = wcat2.shape[1]
    o = o2 // 2

    def body(s_ref, d_ref, xr_ref, b1_ref, w_ref, zl_ref, zr_ref, inv_ref):
        dp = d_ref[...]
        deg = dp[0, :, 0:1] + dp[1, :, 0:1]
        inv = 1.0 / jnp.maximum(deg, 1.0)
        sp = s_ref[...]
        seg = sp[0] + sp[1]
        z1 = jnp.maximum(seg * inv + b1_ref[...] + xr_ref[...], 0.0)
        zw = jnp.dot(z1, w_ref[...], preferred_element_type=jnp.float32)
        zl_ref[...] = zw[:, :o]
        zr_ref[...] = zw[:, o:]
        inv_ref[...] = inv

    return pl.pallas_call(
        body,
        out_shape=[jax.ShapeDtypeStruct((n, o), jnp.float32),
                   jax.ShapeDtypeStruct((n, o), jnp.float32),
                   jax.ShapeDtypeStruct((n, 1), jnp.float32)],
    )(seg1p, degp, xr, b1r, wcat2)


def _final_nodes(seg2p, inv, zr, b2r):
    n, o = zr.shape

    def body(s_ref, i_ref, zr_ref, b2_ref, z_ref):
        sp = s_ref[...]
        z_ref[...] = (sp[0] + sp[1]) * i_ref[...] + b2_ref[...] + zr_ref[...]

    return pl.pallas_call(
        body,
        out_shape=jax.ShapeDtypeStruct((n, o), jnp.float32),
    )(seg2p, inv, zr, b2r)


def _mlp(zu, zv, pfp, w1s, wpf, b1m, w2t, b2m, w3p, b3p, bp=2048):
    pp, o = zu.shape
    pfd = pfp.shape[1]
    mh = w1s.shape[1]
    mh2 = w2t.shape[1]
    ow = w3p.shape[1]
    grid = pp // bp

    def body(zu_ref, zv_ref, pf_ref, w1_ref, wp_ref, b1_ref, w2_ref, b2_ref,
             w3_ref, b3_ref, out_ref):
        a = zu_ref[...]
        bv = zv_ref[...]
        ad = jnp.abs(a - bv)
        pr = a * bv
        h1 = (jnp.dot(a, w1_ref[0:o], preferred_element_type=jnp.float32)
              + jnp.dot(bv, w1_ref[o:2 * o], preferred_element_type=jnp.float32)
              + jnp.dot(ad, w1_ref[2 * o:3 * o], preferred_element_type=jnp.float32)
              + jnp.dot(pr, w1_ref[3 * o:4 * o], preferred_element_type=jnp.float32)
              + jnp.dot(pf_ref[...], wp_ref[...], preferred_element_type=jnp.float32)
              + b1_ref[...])
        h1 = jnp.maximum(h1, 0.0)
        h2 = jnp.maximum(jnp.dot(h1, w2_ref[...], preferred_element_type=jnp.float32)
                         + b2_ref[...], 0.0)
        out_ref[...] = jnp.dot(h2, w3_ref[...], preferred_element_type=jnp.float32) + b3_ref[...]

    return pl.pallas_call(
        body,
        grid=(grid,),
        in_specs=[
            pl.BlockSpec((bp, o), lambda i: (i, 0)),
            pl.BlockSpec((bp, o), lambda i: (i, 0)),
            pl.BlockSpec((bp, pfd), lambda i: (i, 0)),
            pl.BlockSpec((4 * o, mh), lambda i: (0, 0)),
            pl.BlockSpec((pfd, mh), lambda i: (0, 0)),
            pl.BlockSpec((1, mh), lambda i: (0, 0)),
            pl.BlockSpec((mh, mh2), lambda i: (0, 0)),
            pl.BlockSpec((1, mh2), lambda i: (0, 0)),
            pl.BlockSpec((mh2, ow), lambda i: (0, 0)),
            pl.BlockSpec((1, ow), lambda i: (0, 0)),
        ],
        out_specs=pl.BlockSpec((bp, ow), lambda i: (i, 0)),
        out_shape=jax.ShapeDtypeStruct((pp, ow), jnp.float32),
    )(zu, zv, pfp, w1s, wpf, b1m, w2t, b2m, w3p, b3p)


def kernel(x, edge_index, edge_label_index, pair_feats,
           Wl1, Wr1, b1, Wl2, Wr2, b2, Wm1, bm1, Wm2, bm2, Wm3, bm3):
    n = x.shape[0]
    p = edge_label_index.shape[1]
    h = Wl1.shape[0]
    o = Wl2.shape[0]
    mh = Wm1.shape[0]
    mh2 = Wm2.shape[0]
    pfd = pair_feats.shape[1]

    src = edge_index[0]
    dst = edge_index[1]

    nunit = _NS * 8
    n_pad = ((n + nunit - 1) // nunit) * nunit
    xp = jnp.pad(x, ((0, n_pad - n), (0, 0)))

    wcat1 = jnp.concatenate([Wl1.T, Wr1.T], axis=1)
    xl, xr = _encode_in(xp, wcat1)

    epw = edge_index.shape[1] // _NW
    b_e = _pick_chunk(epw)
    zeros_h = jnp.zeros((n_pad, h), jnp.float32)
    zeros_1 = jnp.zeros((n_pad, 16), jnp.float32)
    ones_b = jnp.ones((b_e, 16), jnp.float32)
    seg1p, degp = _seg_sum_sc(xl, src, dst, zeros_h, zeros_1, ones_b,
                              with_deg=True)

    wcat2 = jnp.concatenate([Wl2.T, Wr2.T], axis=1)
    zl, zr, inv = _mid(seg1p, degp, xr, b1.reshape(1, h), wcat2)

    zeros_o = jnp.zeros((n_pad, o), jnp.float32)
    seg2p = _seg_sum_sc(zl, src, dst, zeros_o, None, None, with_deg=False)

    z = _final_nodes(seg2p, inv, zr, b2.reshape(1, o))

    unit = _NW * 128
    pp = ((p + unit - 1) // unit) * unit
    u = jnp.pad(edge_label_index[0], (0, pp - p))
    v = jnp.pad(edge_label_index[1], (0, pp - p))
    zu, zv = _pair_gather_sc(z, u, v)

    pfpad = 8
    pfp = jnp.pad(pair_feats, ((0, pp - p), (0, pfpad - pfd)))
    w1s = Wm1.T[:4 * o]
    wpf = jnp.pad(Wm1.T[4 * o:], ((0, pfpad - pfd), (0, 0)))
    w3p = jnp.pad(Wm3.T, ((0, 0), (0, 7)))
    b3p = jnp.pad(bm3.reshape(1, 1), ((0, 0), (0, 7)))
    out8 = _mlp(zu, zv, pfp, w1s, wpf, bm1.reshape(1, mh), Wm2.T,
                bm2.reshape(1, mh2), w3p, b3p)
    return out8[:p, 0]
